# baseline (speedup 1.0000x reference)
import functools

import jax
import jax.numpy as jnp
from jax import lax
from jax.experimental import pallas as pl
from jax.experimental.pallas import tpu as pltpu
from jax.experimental.pallas import tpu_sc as plsc

N_NODES = 10000
N_EDGES = 160000
D_IN = 256
D_HID = 512
N_CLASSES = 40
PC = 48
DHALF = 128
XW = DHALF + 16

NCORES = 2
NSUB = 16
N_PAD = 10240
STRIPE = N_PAD // NSUB

E_PER_SUB1 = N_EDGES // NSUB
CH1 = 40
NCH1 = E_PER_SUB1 // CH1

E_PER_SUB2 = N_EDGES // (2 * NSUB)
CH2 = 40
NCH2 = E_PER_SUB2 // CH2

RB = 2000
_F32 = jnp.float32


def _sage_sc_mesh():
    return plsc.VectorSubcoreMesh(core_axis_name="c", subcore_axis_name="s")


@functools.partial(
    pl.kernel,
    out_type=jax.ShapeDtypeStruct((2, N_PAD, XW), _F32),
    mesh=_sage_sc_mesh(),
    compiler_params=pltpu.CompilerParams(use_tc_tiling_on_sc=False),
    scratch_types=[
        pltpu.VMEM_SHARED((N_PAD, XW), _F32),
        pltpu.VMEM((E_PER_SUB1,), jnp.int32),
        pltpu.VMEM((NCH1, CH1), jnp.int32),
        pltpu.VMEM((2, CH1, XW), _F32),
        pltpu.SemaphoreType.DMA,
    ],
)
def _k1_aggregate(xs2, srcr, dstr, zrow,
                  agg_out,
                  acc_sh, src_v, dst_v, rows_v, sem):
    cid = lax.axis_index("c")
    sid = lax.axis_index("s")
    r0 = sid * STRIPE

    pltpu.sync_copy(zrow, acc_sh.at[pl.ds(r0, STRIPE)])

    pltpu.sync_copy(srcr.at[sid], src_v)
    pltpu.sync_copy(dstr.at[sid], dst_v)

    @pl.when(cid == 1)
    def _():
        def adj(i, _):
            src_v[pl.ds(i * 16, 16)] = src_v[pl.ds(i * 16, 16)] + N_NODES
            return 0
        lax.fori_loop(0, E_PER_SUB1 // 16, adj, 0)

    plsc.subcore_barrier()

    pltpu.async_copy(xs2.at[src_v.at[pl.ds(0, CH1)]], rows_v.at[0], sem)

    def pair(t, _):
        for b in range(2):
            j = 2 * t + b
            pltpu.make_async_copy(xs2.at[src_v.at[pl.ds(j * CH1, CH1)]], rows_v.at[b], sem).wait()

            @pl.when(j + 1 < NCH1)
            def _():
                pltpu.async_copy(xs2.at[src_v.at[pl.ds((j + 1) * CH1, CH1)]], rows_v.at[1 - b], sem)

            pltpu.sync_copy(rows_v.at[b], acc_sh.at[dst_v.at[j]], add=True)
        return 0

    lax.fori_loop(0, NCH1 // 2, pair, 0)
    plsc.subcore_barrier()

    pltpu.sync_copy(acc_sh.at[pl.ds(r0, STRIPE)],
                    agg_out.at[cid, pl.ds(r0, STRIPE)])


@functools.partial(
    pl.kernel,
    out_type=jax.ShapeDtypeStruct((2, N_PAD, PC), _F32),
    mesh=_sage_sc_mesh(),
    compiler_params=pltpu.CompilerParams(use_tc_tiling_on_sc=False),
    scratch_types=[
        pltpu.VMEM_SHARED((N_PAD, PC), _F32),
        pltpu.VMEM((NCH2, CH2), jnp.int32),
        pltpu.VMEM((NCH2, CH2), jnp.int32),
        pltpu.VMEM((2, CH2, PC), _F32),
        pltpu.SemaphoreType.DMA,
    ],
)
def _k3_aggregate(p_hbm, srcr, dstr, zrow,
                  agg_out,
                  acc_sh, src_v, dst_v, rows_v, sem):
    cid = lax.axis_index("c")
    sid = lax.axis_index("s")
    r0 = sid * STRIPE

    pltpu.sync_copy(zrow, acc_sh.at[pl.ds(r0, STRIPE)])
    pltpu.sync_copy(srcr.at[cid, sid], src_v)
    pltpu.sync_copy(dstr.at[cid, sid], dst_v)
    plsc.subcore_barrier()

    pltpu.async_copy(p_hbm.at[src_v.at[0]], rows_v.at[0], sem)

    def pair(t, _):
        for b in range(2):
            j = 2 * t + b
            pltpu.make_async_copy(p_hbm.at[src_v.at[j]], rows_v.at[b], sem).wait()

            @pl.when(j + 1 < NCH2)
            def _():
                pltpu.async_copy(p_hbm.at[src_v.at[j + 1]], rows_v.at[1 - b], sem)

            pltpu.sync_copy(rows_v.at[b], acc_sh.at[dst_v.at[j]], add=True)
        return 0

    lax.fori_loop(0, NCH2 // 2, pair, 0)
    pltpu.make_async_copy(p_hbm.at[src_v.at[NCH2 - 1]], rows_v.at[0], sem).wait()
    pltpu.sync_copy(rows_v.at[0], acc_sh.at[dst_v.at[NCH2 - 1]], add=True)
    plsc.subcore_barrier()

    pltpu.sync_copy(acc_sh.at[pl.ds(r0, STRIPE)],
                    agg_out.at[cid, pl.ds(r0, STRIPE)])


def _k0_body(wl2, wr2, b2r, wcp, bcp, wlc_o, wrc_o, bcc_o):
    wlc_o[...] = jnp.dot(wl2[...], wcp[...], preferred_element_type=_F32)
    wrc_o[...] = jnp.dot(wr2[...], wcp[...], preferred_element_type=_F32)
    bcc_o[...] = jnp.dot(b2r[...], wcp[...], preferred_element_type=_F32) + bcp[...]


def _weight_fold(Wl2, Wr2, b2r, Wcp, bcp):
    return pl.pallas_call(
        _k0_body,
        out_shape=[
            jax.ShapeDtypeStruct((D_HID, PC), _F32),
            jax.ShapeDtypeStruct((D_HID, PC), _F32),
            jax.ShapeDtypeStruct((1, PC), _F32),
        ],
    )(Wl2, Wr2, b2r, Wcp, bcp)


def _k2_body(agg, x, wl1, wr1, b1, wlc, wrc, bcc, p_o, q_o):
    inv = 1.0 / jnp.maximum(agg[0][:, DHALF:DHALF + 1], 1.0)
    mlo = agg[0][:, 0:DHALF] * inv
    mhi = agg[1][:, 0:DHALF] * inv
    h = (jnp.dot(mlo, wl1[0:DHALF, :], preferred_element_type=_F32)
         + jnp.dot(mhi, wl1[DHALF:D_IN, :], preferred_element_type=_F32)
         + jnp.dot(x[...], wr1[...], preferred_element_type=_F32)
         + b1[...])
    h = jnp.maximum(h, 0.0)
    p_o[...] = jnp.dot(h, wlc[...], preferred_element_type=_F32)
    q_o[...] = jnp.dot(h, wrc[...], preferred_element_type=_F32) + bcc[...]


def _layer1_tc(agg1, x, Wl1, Wr1, b1r, WlC, WrC, bcc):
    nblk = N_NODES // RB
    full = lambda i: (0, 0)
    return pl.pallas_call(
        _k2_body,
        grid=(nblk,),
        in_specs=[
            pl.BlockSpec((2, RB, XW), lambda i: (0, i, 0)),
            pl.BlockSpec((RB, D_IN), lambda i: (i, 0)),
            pl.BlockSpec((D_IN, D_HID), full),
            pl.BlockSpec((D_IN, D_HID), full),
            pl.BlockSpec((1, D_HID), full),
            pl.BlockSpec((D_HID, PC), full),
            pl.BlockSpec((D_HID, PC), full),
            pl.BlockSpec((1, PC), full),
        ],
        out_specs=[
            pl.BlockSpec((RB, PC), lambda i: (i, 0)),
            pl.BlockSpec((RB, PC), lambda i: (i, 0)),
        ],
        out_shape=[
            jax.ShapeDtypeStruct((N_NODES, PC), _F32),
            jax.ShapeDtypeStruct((N_NODES, PC), _F32),
        ],
    )(agg1, x, Wl1, Wr1, b1r, WlC, WrC, bcc)


def _k4_body(agg2, agg1, q, out):
    inv = 1.0 / jnp.maximum(agg1[0][:, DHALF:DHALF + 1], 1.0)
    o = (agg2[0] + agg2[1]) * inv + q[...]
    out[...] = o[:, 0:N_CLASSES]


def _finalize_tc(agg2, agg1, q):
    nblk = N_NODES // RB
    return pl.pallas_call(
        _k4_body,
        grid=(nblk,),
        in_specs=[
            pl.BlockSpec((2, RB, PC), lambda i: (0, i, 0)),
            pl.BlockSpec((1, RB, XW), lambda i: (0, i, 0)),
            pl.BlockSpec((RB, PC), lambda i: (i, 0)),
        ],
        out_specs=pl.BlockSpec((RB, N_CLASSES), lambda i: (i, 0)),
        out_shape=jax.ShapeDtypeStruct((N_NODES, N_CLASSES), _F32),
    )(agg2, agg1, q)


def kernel(x, edge_index, Wl1, Wr1, b1, Wl2, Wr2, b2, Wc, bc):
    src = edge_index[0].astype(jnp.int32)
    dst = edge_index[1].astype(jnp.int32)

    ones_blk = jnp.ones((N_NODES, 16), _F32)
    xs2 = jnp.concatenate(
        [jnp.concatenate([x[:, :DHALF], ones_blk], axis=1),
         jnp.concatenate([x[:, DHALF:], ones_blk], axis=1)], axis=0)

    src1 = src.reshape(NSUB, E_PER_SUB1)
    dst1 = dst.reshape(NSUB, NCH1, CH1)
    src2 = src.reshape(2, NSUB, NCH2, CH2)
    dst2 = dst.reshape(2, NSUB, NCH2, CH2)

    zx = jnp.zeros((STRIPE, XW), _F32)
    zp = jnp.zeros((STRIPE, PC), _F32)

    b1r = b1.reshape(1, D_HID)
    b2r = b2.reshape(1, D_HID)
    Wcp = jnp.pad(Wc, ((0, 0), (0, PC - N_CLASSES)))
    bcp = jnp.pad(bc, (0, PC - N_CLASSES)).reshape(1, PC)

    agg1 = _k1_aggregate(xs2, src1, dst1, zx)
    WlC, WrC, bcc = _weight_fold(Wl2, Wr2, b2r, Wcp, bcp)
    p, q = _layer1_tc(agg1, x, Wl1, Wr1, b1r, WlC, WrC, bcc)
    agg2 = _k3_aggregate(p, src2, dst2, zp)
    return _finalize_tc(agg2, agg1, q)


# speedup vs baseline: 1.4705x; 1.4705x over previous
import functools

import jax
import jax.numpy as jnp
from jax import lax
from jax.experimental import pallas as pl
from jax.experimental.pallas import tpu as pltpu
from jax.experimental.pallas import tpu_sc as plsc

N_NODES = 10000
N_EDGES = 160000
D_IN = 256
D_HID = 512
N_CLASSES = 40
PC = 48
XW = D_IN + 16

NCORES = 2
NSUB = 16
N_PAD = 10240
STRIPE = N_PAD // NSUB

E_PER_TILE = N_EDGES // (2 * NSUB)
CH1 = 104
NF1 = E_PER_TILE // CH1
TAIL = E_PER_TILE - NF1 * CH1

RB = 2000
_F32 = jnp.float32
_BF16 = jnp.bfloat16


def _sage_sc_mesh():
    return plsc.VectorSubcoreMesh(core_axis_name="c", subcore_axis_name="s")


def _sc_edge_loop(table, src_v, dst_v, dstt_v, rows_v, acc_sh, sem, ch, nf):
    pltpu.async_copy(table.at[src_v.at[pl.ds(0, ch)]], rows_v.at[0], sem)

    def pair(t, _):
        for b in range(2):
            k = 2 * t + b
            pltpu.make_async_copy(
                table.at[src_v.at[pl.ds(k * ch, ch)]], rows_v.at[b], sem
            ).wait()

            @pl.when(k + 1 < nf)
            def _():
                pltpu.async_copy(
                    table.at[src_v.at[pl.ds((k + 1) * ch, ch)]],
                    rows_v.at[1 - b], sem)

            @pl.when(k + 1 == nf)
            def _():
                pltpu.async_copy(
                    table.at[src_v.at[pl.ds(nf * ch, TAIL)]],
                    rows_v.at[1 - b, pl.ds(0, TAIL)], sem)

            pltpu.sync_copy(rows_v.at[b], acc_sh.at[dst_v.at[k]], add=True)
        return 0

    lax.fori_loop(0, nf // 2, pair, 0)
    pltpu.make_async_copy(
        table.at[src_v.at[pl.ds(nf * ch, TAIL)]],
        rows_v.at[0, pl.ds(0, TAIL)], sem).wait()
    pltpu.sync_copy(rows_v.at[0, pl.ds(0, TAIL)],
                    acc_sh.at[dstt_v.at[0]], add=True)


@functools.partial(
    pl.kernel,
    out_type=jax.ShapeDtypeStruct((2, N_PAD, XW), _BF16),
    mesh=_sage_sc_mesh(),
    compiler_params=pltpu.CompilerParams(use_tc_tiling_on_sc=False),
    scratch_types=[
        pltpu.VMEM_SHARED((N_PAD, XW), _BF16),
        pltpu.VMEM((E_PER_TILE,), jnp.int32),
        pltpu.VMEM((NF1, CH1), jnp.int32),
        pltpu.VMEM((1, TAIL), jnp.int32),
        pltpu.VMEM((2, CH1, XW), _BF16),
        pltpu.SemaphoreType.DMA,
    ],
)
def _k1_aggregate(xb, srcr, dstr, dstt, zrow,
                  agg_out,
                  acc_sh, src_v, dst_v, dstt_v, rows_v, sem):
    cid = lax.axis_index("c")
    sid = lax.axis_index("s")
    r0 = sid * STRIPE

    pltpu.sync_copy(zrow, acc_sh.at[pl.ds(r0, STRIPE)])
    pltpu.sync_copy(srcr.at[cid, sid], src_v)
    pltpu.sync_copy(dstr.at[cid, sid], dst_v)
    pltpu.sync_copy(dstt.at[cid, sid], dstt_v)
    plsc.subcore_barrier()

    _sc_edge_loop(xb, src_v, dst_v, dstt_v, rows_v, acc_sh, sem, CH1, NF1)

    plsc.subcore_barrier()
    pltpu.sync_copy(acc_sh.at[pl.ds(r0, STRIPE)],
                    agg_out.at[cid, pl.ds(r0, STRIPE)])


@functools.partial(
    pl.kernel,
    out_type=jax.ShapeDtypeStruct((2, N_PAD, PC), _F32),
    mesh=_sage_sc_mesh(),
    compiler_params=pltpu.CompilerParams(use_tc_tiling_on_sc=False),
    scratch_types=[
        pltpu.VMEM_SHARED((N_PAD, PC), _F32),
        pltpu.VMEM((E_PER_TILE,), jnp.int32),
        pltpu.VMEM((NF1, CH1), jnp.int32),
        pltpu.VMEM((1, TAIL), jnp.int32),
        pltpu.VMEM((2, CH1, PC), _F32),
        pltpu.SemaphoreType.DMA,
    ],
)
def _k3_aggregate(p_hbm, srcr, dstr, dstt, zrow,
                  agg_out,
                  acc_sh, src_v, dst_v, dstt_v, rows_v, sem):
    cid = lax.axis_index("c")
    sid = lax.axis_index("s")
    r0 = sid * STRIPE

    pltpu.sync_copy(zrow, acc_sh.at[pl.ds(r0, STRIPE)])
    pltpu.sync_copy(srcr.at[cid, sid], src_v)
    pltpu.sync_copy(dstr.at[cid, sid], dst_v)
    pltpu.sync_copy(dstt.at[cid, sid], dstt_v)
    plsc.subcore_barrier()

    _sc_edge_loop(p_hbm, src_v, dst_v, dstt_v, rows_v, acc_sh, sem, CH1, NF1)

    plsc.subcore_barrier()
    pltpu.sync_copy(acc_sh.at[pl.ds(r0, STRIPE)],
                    agg_out.at[cid, pl.ds(r0, STRIPE)])


def _k2_body(agg, x, wl1, wr1, b1, wl2, wr2, b2r, wcp, bcp,
             p_o, q_o, wlc_s, wrc_s, bcc_s):
    @pl.when(pl.program_id(0) == 0)
    def _():
        wlc_s[...] = jnp.dot(wl2[...], wcp[...], preferred_element_type=_F32)
        wrc_s[...] = jnp.dot(wr2[...], wcp[...], preferred_element_type=_F32)
        bcc_s[...] = jnp.dot(b2r[...], wcp[...],
                             preferred_element_type=_F32) + bcp[...]

    s = agg[0].astype(_F32) + agg[1].astype(_F32)
    inv = 1.0 / jnp.maximum(s[:, D_IN:D_IN + 1], 1.0)
    m = s[:, 0:D_IN] * inv
    h = (jnp.dot(m, wl1[...], preferred_element_type=_F32)
         + jnp.dot(x[...], wr1[...], preferred_element_type=_F32)
         + b1[...])
    h = jnp.maximum(h, 0.0)
    p_o[...] = jnp.dot(h, wlc_s[...], preferred_element_type=_F32)
    q_o[...] = jnp.dot(h, wrc_s[...], preferred_element_type=_F32) + bcc_s[...]


def _layer1_tc(agg1, x, Wl1, Wr1, b1r, Wl2, Wr2, b2r, Wcp, bcp):
    nblk = N_NODES // RB
    full = lambda i: (0, 0)
    return pl.pallas_call(
        _k2_body,
        grid=(nblk,),
        in_specs=[
            pl.BlockSpec((2, RB, XW), lambda i: (0, i, 0)),
            pl.BlockSpec((RB, D_IN), lambda i: (i, 0)),
            pl.BlockSpec((D_IN, D_HID), full),
            pl.BlockSpec((D_IN, D_HID), full),
            pl.BlockSpec((1, D_HID), full),
            pl.BlockSpec((D_HID, D_HID), full),
            pl.BlockSpec((D_HID, D_HID), full),
            pl.BlockSpec((1, D_HID), full),
            pl.BlockSpec((D_HID, PC), full),
            pl.BlockSpec((1, PC), full),
        ],
        out_specs=[
            pl.BlockSpec((RB, PC), lambda i: (i, 0)),
            pl.BlockSpec((RB, PC), lambda i: (i, 0)),
        ],
        out_shape=[
            jax.ShapeDtypeStruct((N_NODES, PC), _F32),
            jax.ShapeDtypeStruct((N_NODES, PC), _F32),
        ],
        scratch_shapes=[
            pltpu.VMEM((D_HID, PC), _F32),
            pltpu.VMEM((D_HID, PC), _F32),
            pltpu.VMEM((1, PC), _F32),
        ],
    )(agg1, x, Wl1, Wr1, b1r, Wl2, Wr2, b2r, Wcp, bcp)


def _k4_body(agg2, agg1, q, out):
    s = agg1[0].astype(_F32) + agg1[1].astype(_F32)
    inv = 1.0 / jnp.maximum(s[:, D_IN:D_IN + 1], 1.0)
    o = (agg2[0] + agg2[1]) * inv + q[...]
    out[...] = o[:, 0:N_CLASSES]


def _finalize_tc(agg2, agg1, q):
    nblk = N_NODES // RB
    return pl.pallas_call(
        _k4_body,
        grid=(nblk,),
        in_specs=[
            pl.BlockSpec((2, RB, PC), lambda i: (0, i, 0)),
            pl.BlockSpec((2, RB, XW), lambda i: (0, i, 0)),
            pl.BlockSpec((RB, PC), lambda i: (i, 0)),
        ],
        out_specs=pl.BlockSpec((RB, N_CLASSES), lambda i: (i, 0)),
        out_shape=jax.ShapeDtypeStruct((N_NODES, N_CLASSES), _F32),
    )(agg2, agg1, q)


def kernel(x, edge_index, Wl1, Wr1, b1, Wl2, Wr2, b2, Wc, bc):
    src = edge_index[0].astype(jnp.int32)
    dst = edge_index[1].astype(jnp.int32)

    xb = jnp.concatenate(
        [x, jnp.ones((N_NODES, 16), _F32)], axis=1).astype(_BF16)

    src3 = src.reshape(2, NSUB, E_PER_TILE)
    dst3 = dst.reshape(2, NSUB, E_PER_TILE)
    dst1f = dst3[:, :, :NF1 * CH1].reshape(2, NSUB, NF1, CH1)
    dstt = dst3[:, :, NF1 * CH1:].reshape(2, NSUB, 1, TAIL)

    zx = jnp.zeros((STRIPE, XW), _BF16)
    zp = jnp.zeros((STRIPE, PC), _F32)

    b1r = b1.reshape(1, D_HID)
    b2r = b2.reshape(1, D_HID)
    Wcp = jnp.pad(Wc, ((0, 0), (0, PC - N_CLASSES)))
    bcp = jnp.pad(bc, (0, PC - N_CLASSES)).reshape(1, PC)

    agg1 = _k1_aggregate(xb, src3, dst1f, dstt, zx)
    p, q = _layer1_tc(agg1, x, Wl1, Wr1, b1r, Wl2, Wr2, b2r, Wcp, bcp)
    agg2 = _k3_aggregate(p, src3, dst1f, dstt, zp)
    return _finalize_tc(agg2, agg1, q)


# speedup vs baseline: 1.4903x; 1.0135x over previous
import functools

import jax
import jax.numpy as jnp
from jax import lax
from jax.experimental import pallas as pl
from jax.experimental.pallas import tpu as pltpu
from jax.experimental.pallas import tpu_sc as plsc

N_NODES = 10000
N_EDGES = 160000
D_IN = 256
D_HID = 512
N_CLASSES = 40
PC = 48
XW = D_IN + 16

NCORES = 2
NSUB = 16
N_PAD = 10240
STRIPE = N_PAD // NSUB

E_PER_TILE = N_EDGES // (2 * NSUB)
CH1 = 104
NF1 = E_PER_TILE // CH1
TAIL = E_PER_TILE - NF1 * CH1

RB = 2000
_F32 = jnp.float32
_BF16 = jnp.bfloat16


def _sage_sc_mesh():
    return plsc.VectorSubcoreMesh(core_axis_name="c", subcore_axis_name="s")


def _sc_edge_loop(table, src_v, dst_v, rows_v, acc_sh, sem, ch, nf):
    pltpu.async_copy(table.at[src_v.at[pl.ds(0, ch)]], rows_v.at[0], sem)

    def pair(t, _):
        for b in range(2):
            k = 2 * t + b
            pltpu.make_async_copy(
                table.at[src_v.at[pl.ds(k * ch, ch)]], rows_v.at[b], sem
            ).wait()

            @pl.when(k + 1 < nf)
            def _():
                pltpu.async_copy(
                    table.at[src_v.at[pl.ds((k + 1) * ch, ch)]],
                    rows_v.at[1 - b], sem)

            @pl.when(k + 1 == nf)
            def _():
                pltpu.async_copy(
                    table.at[src_v.at[pl.ds(nf * ch, TAIL)]],
                    rows_v.at[1 - b, pl.ds(0, TAIL)], sem)

            pltpu.sync_copy(rows_v.at[b],
                            acc_sh.at[dst_v.at[pl.ds(k * ch, ch)]], add=True)
        return 0

    lax.fori_loop(0, nf // 2, pair, 0)
    pltpu.make_async_copy(
        table.at[src_v.at[pl.ds(nf * ch, TAIL)]],
        rows_v.at[0, pl.ds(0, TAIL)], sem).wait()
    pltpu.sync_copy(rows_v.at[0, pl.ds(0, TAIL)],
                    acc_sh.at[dst_v.at[pl.ds(nf * ch, TAIL)]], add=True)


@functools.partial(
    pl.kernel,
    out_type=jax.ShapeDtypeStruct((2, N_PAD, XW), _BF16),
    mesh=_sage_sc_mesh(),
    compiler_params=pltpu.CompilerParams(use_tc_tiling_on_sc=False),
    scratch_types=[
        pltpu.VMEM_SHARED((N_PAD, XW), _BF16),
        pltpu.VMEM((E_PER_TILE,), jnp.int32),
        pltpu.VMEM((E_PER_TILE,), jnp.int32),
        pltpu.VMEM((2, CH1, XW), _BF16),
        pltpu.SemaphoreType.DMA,
    ],
)
def _k1_aggregate(xb, srcr, dstr, zrow,
                  agg_out,
                  acc_sh, src_v, dst_v, rows_v, sem):
    cid = lax.axis_index("c")
    sid = lax.axis_index("s")
    r0 = sid * STRIPE

    pltpu.sync_copy(zrow, acc_sh.at[pl.ds(r0, STRIPE)])
    pltpu.sync_copy(srcr.at[cid, sid], src_v)
    pltpu.sync_copy(dstr.at[cid, sid], dst_v)
    plsc.subcore_barrier()

    _sc_edge_loop(xb, src_v, dst_v, rows_v, acc_sh, sem, CH1, NF1)

    plsc.subcore_barrier()
    pltpu.sync_copy(acc_sh.at[pl.ds(r0, STRIPE)],
                    agg_out.at[cid, pl.ds(r0, STRIPE)])


@functools.partial(
    pl.kernel,
    out_type=jax.ShapeDtypeStruct((2, N_PAD, PC), _F32),
    mesh=_sage_sc_mesh(),
    compiler_params=pltpu.CompilerParams(use_tc_tiling_on_sc=False),
    scratch_types=[
        pltpu.VMEM_SHARED((N_PAD, PC), _F32),
        pltpu.VMEM((E_PER_TILE,), jnp.int32),
        pltpu.VMEM((E_PER_TILE,), jnp.int32),
        pltpu.VMEM((2, CH1, PC), _F32),
        pltpu.SemaphoreType.DMA,
    ],
)
def _k3_aggregate(p_hbm, srcr, dstr, zrow,
                  agg_out,
                  acc_sh, src_v, dst_v, rows_v, sem):
    cid = lax.axis_index("c")
    sid = lax.axis_index("s")
    r0 = sid * STRIPE

    pltpu.sync_copy(zrow, acc_sh.at[pl.ds(r0, STRIPE)])
    pltpu.sync_copy(srcr.at[cid, sid], src_v)
    pltpu.sync_copy(dstr.at[cid, sid], dst_v)
    plsc.subcore_barrier()

    _sc_edge_loop(p_hbm, src_v, dst_v, rows_v, acc_sh, sem, CH1, NF1)

    plsc.subcore_barrier()
    pltpu.sync_copy(acc_sh.at[pl.ds(r0, STRIPE)],
                    agg_out.at[cid, pl.ds(r0, STRIPE)])


def _k2_body(agg, x, wl1, wr1, b1, wl2, wr2, b2r, wcp, bcp,
             p_o, q_o, wlc_s, wrc_s, bcc_s):
    @pl.when(pl.program_id(0) == 0)
    def _():
        wlc_s[...] = jnp.dot(wl2[...], wcp[...], preferred_element_type=_F32)
        wrc_s[...] = jnp.dot(wr2[...], wcp[...], preferred_element_type=_F32)
        bcc_s[...] = jnp.dot(b2r[...], wcp[...],
                             preferred_element_type=_F32) + bcp[...]

    s = agg[0].astype(_F32) + agg[1].astype(_F32)
    inv = 1.0 / jnp.maximum(s[:, D_IN:D_IN + 1], 1.0)
    m = s[:, 0:D_IN] * inv
    h = (jnp.dot(m, wl1[...], preferred_element_type=_F32)
         + jnp.dot(x[...], wr1[...], preferred_element_type=_F32)
         + b1[...])
    h = jnp.maximum(h, 0.0)
    p_o[...] = jnp.dot(h, wlc_s[...], preferred_element_type=_F32)
    q_o[...] = jnp.dot(h, wrc_s[...], preferred_element_type=_F32) + bcc_s[...]


def _layer1_tc(agg1, x, Wl1, Wr1, b1r, Wl2, Wr2, b2r, Wcp, bcp):
    nblk = N_NODES // RB
    full = lambda i: (0, 0)
    return pl.pallas_call(
        _k2_body,
        grid=(nblk,),
        in_specs=[
            pl.BlockSpec((2, RB, XW), lambda i: (0, i, 0)),
            pl.BlockSpec((RB, D_IN), lambda i: (i, 0)),
            pl.BlockSpec((D_IN, D_HID), full),
            pl.BlockSpec((D_IN, D_HID), full),
            pl.BlockSpec((1, D_HID), full),
            pl.BlockSpec((D_HID, D_HID), full),
            pl.BlockSpec((D_HID, D_HID), full),
            pl.BlockSpec((1, D_HID), full),
            pl.BlockSpec((D_HID, PC), full),
            pl.BlockSpec((1, PC), full),
        ],
        out_specs=[
            pl.BlockSpec((RB, PC), lambda i: (i, 0)),
            pl.BlockSpec((RB, PC), lambda i: (i, 0)),
        ],
        out_shape=[
            jax.ShapeDtypeStruct((N_NODES, PC), _F32),
            jax.ShapeDtypeStruct((N_NODES, PC), _F32),
        ],
        scratch_shapes=[
            pltpu.VMEM((D_HID, PC), _F32),
            pltpu.VMEM((D_HID, PC), _F32),
            pltpu.VMEM((1, PC), _F32),
        ],
    )(agg1, x, Wl1, Wr1, b1r, Wl2, Wr2, b2r, Wcp, bcp)


def _k4_body(agg2, agg1, q, out):
    s = agg1[0].astype(_F32) + agg1[1].astype(_F32)
    inv = 1.0 / jnp.maximum(s[:, D_IN:D_IN + 1], 1.0)
    o = (agg2[0] + agg2[1]) * inv + q[...]
    out[...] = o[:, 0:N_CLASSES]


def _finalize_tc(agg2, agg1, q):
    nblk = N_NODES // RB
    return pl.pallas_call(
        _k4_body,
        grid=(nblk,),
        in_specs=[
            pl.BlockSpec((2, RB, PC), lambda i: (0, i, 0)),
            pl.BlockSpec((2, RB, XW), lambda i: (0, i, 0)),
            pl.BlockSpec((RB, PC), lambda i: (i, 0)),
        ],
        out_specs=pl.BlockSpec((RB, N_CLASSES), lambda i: (i, 0)),
        out_shape=jax.ShapeDtypeStruct((N_NODES, N_CLASSES), _F32),
    )(agg2, agg1, q)


def kernel(x, edge_index, Wl1, Wr1, b1, Wl2, Wr2, b2, Wc, bc):
    src = edge_index[0].astype(jnp.int32)
    dst = edge_index[1].astype(jnp.int32)

    xb = jnp.concatenate(
        [x, jnp.ones((N_NODES, 16), _F32)], axis=1).astype(_BF16)

    src3 = src.reshape(2, NSUB, E_PER_TILE)
    dst3 = dst.reshape(2, NSUB, E_PER_TILE)

    zx = jnp.zeros((STRIPE, XW), _BF16)
    zp = jnp.zeros((STRIPE, PC), _F32)

    b1r = b1.reshape(1, D_HID)
    b2r = b2.reshape(1, D_HID)
    Wcp = jnp.pad(Wc, ((0, 0), (0, PC - N_CLASSES)))
    bcp = jnp.pad(bc, (0, PC - N_CLASSES)).reshape(1, PC)

    agg1 = _k1_aggregate(xb, src3, dst3, zx)
    p, q = _layer1_tc(agg1, x, Wl1, Wr1, b1r, Wl2, Wr2, b2r, Wcp, bcp)
    agg2 = _k3_aggregate(p, src3, dst3, zp)
    return _finalize_tc(agg2, agg1, q)
